# Initial kernel scaffold; baseline (speedup 1.0000x reference)
import jax
import jax.numpy as jnp
from jax.experimental import pallas as pl


def kernel(cands, anchors, W_dec, W_enc, neighbor_wts):
    raise NotImplementedError("write your pallas kernel here")


# speedup vs baseline: 39.4397x; 39.4397x over previous
import jax
import jax.numpy as jnp
from jax.experimental import pallas as pl
from jax.experimental.pallas import tpu as pltpu

LEVELS = 3
K_NEIGH = 4
N_CANDS = 4096
M_ANCH = 4096
D_LAT = 128
D_DATA = 512
STRIP = 256
N_STRIPS = N_CANDS // STRIP

_F32 = jnp.float32
_DN = (((1,), (0,)), ((), ()))
_DN_RHS_T = (((1,), (1,)), ((), ()))


def _main_kernel(cands_ref, anchors_ref, wd_ref, we_ref, wts_ref,
                 score_ref, wd_bf, we_bf, d1_bf, a_sc, r_sc, a_bf, r_bf,
                 an_sc):
    l = pl.program_id(0)
    i = pl.program_id(1)
    bf = jnp.bfloat16

    @pl.when(i == 0)
    def _new_level():
        @pl.when(l == 0)
        def _init():
            wd_bf[...] = wd_ref[...].astype(bf)
            we_bf[...] = we_ref[...].astype(bf)
            a_sc[...] = anchors_ref[...]
            r_sc[...] = cands_ref[...]

        @pl.when(l > 0)
        def _advance():
            d1_bf[...] = jax.lax.dot_general(
                a_bf[...], wd_bf[...], _DN,
                preferred_element_type=_F32).astype(bf)
            a_sc[...] = jax.lax.dot_general(
                d1_bf[...], we_bf[...], _DN, preferred_element_type=_F32)
            d1_bf[...] = jax.lax.dot_general(
                r_bf[...], wd_bf[...], _DN,
                preferred_element_type=_F32).astype(bf)
            r_sc[...] = jax.lax.dot_general(
                d1_bf[...], we_bf[...], _DN, preferred_element_type=_F32)

        a = a_sc[...]
        a_bf[...] = a.astype(bf)
        r_bf[...] = r_sc[...].astype(bf)
        an_sc[...] = jnp.sum(a * a, axis=1, keepdims=True)

    ab = jax.lax.dot_general(a_bf[...], r_bf[pl.ds(i * STRIP, STRIP), :],
                             _DN_RHS_T, preferred_element_type=_F32)
    r = r_sc[pl.ds(i * STRIP, STRIP), :]
    ones = jnp.ones((1, D_LAT), _F32)
    rn = jax.lax.dot_general(ones, r * r, _DN_RHS_T,
                             preferred_element_type=_F32,
                             precision=jax.lax.Precision.HIGHEST)
    d2 = jnp.maximum(an_sc[...] + rn - 2.0 * ab, 1e-12)

    iota = jax.lax.broadcasted_iota(jnp.int32, (M_ANCH, STRIP), 0)
    big = jnp.float32(3.0e38)
    big_i = jnp.int32(2 ** 30)
    acc = jnp.zeros((1, STRIP), _F32)
    for k in range(K_NEIGH):
        m = jnp.min(d2, axis=0, keepdims=True)
        w_k = wts_ref[0, k].astype(bf).astype(_F32)
        acc = acc + w_k * jnp.sqrt(m).astype(bf).astype(_F32)
        if k < K_NEIGH - 1:
            first = jnp.min(jnp.where(d2 <= m, iota, big_i),
                            axis=0, keepdims=True)
            d2 = jnp.where(iota == first, big, d2)
    score_ref[0] = acc


def _finish_kernel(score_ref, cands_ref, gold_ref, acc_ref, picks_ref):
    i = pl.program_id(0)
    s = score_ref[...]
    g_row = jnp.logical_and(
        jnp.logical_and(s[1:2] < s[0:1], s[2:3] < s[1:2]),
        s[3:4] < s[2:3])
    gf = g_row.astype(_F32)
    ii = jax.lax.broadcasted_iota(jnp.int32, (STRIP, STRIP), 0)
    jj = jax.lax.broadcasted_iota(jnp.int32, (STRIP, STRIP), 1)
    eye = (ii == jj).astype(_F32)
    gcol = jax.lax.dot_general(eye, gf, _DN_RHS_T,
                               preferred_element_type=_F32, precision=jax.lax.Precision.HIGHEST)
    picks_ref[...] = cands_ref[...] * gcol
    gold_ref[0] = g_row.astype(jnp.int32)

    @pl.when(i == 0)
    def _zero():
        acc_ref[...] = jnp.zeros((1, 1), _F32)

    acc_ref[...] += jnp.sum(gf, keepdims=True) / N_CANDS


def kernel(cands, anchors, W_dec, W_enc, neighbor_wts):
    wts2d = jnp.reshape(neighbor_wts, (1, K_NEIGH))
    score3d = pl.pallas_call(
        _main_kernel,
        grid=(LEVELS + 1, N_STRIPS),
        in_specs=[
            pl.BlockSpec((N_CANDS, D_LAT), lambda l, i: (0, 0)),
            pl.BlockSpec((M_ANCH, D_LAT), lambda l, i: (0, 0)),
            pl.BlockSpec((D_LAT, D_DATA), lambda l, i: (0, 0)),
            pl.BlockSpec((D_DATA, D_LAT), lambda l, i: (0, 0)),
            pl.BlockSpec((1, K_NEIGH), lambda l, i: (0, 0)),
        ],
        out_specs=pl.BlockSpec((1, 1, STRIP), lambda l, i: (l * N_STRIPS + i, 0, 0)),
        out_shape=jax.ShapeDtypeStruct(((LEVELS + 1) * N_STRIPS, 1, STRIP), _F32),
        scratch_shapes=[
            pltpu.VMEM((D_LAT, D_DATA), jnp.bfloat16),
            pltpu.VMEM((D_DATA, D_LAT), jnp.bfloat16),
            pltpu.VMEM((M_ANCH, D_DATA), jnp.bfloat16),
            pltpu.VMEM((M_ANCH, D_LAT), _F32),
            pltpu.VMEM((N_CANDS, D_LAT), _F32),
            pltpu.VMEM((M_ANCH, D_LAT), jnp.bfloat16),
            pltpu.VMEM((N_CANDS, D_LAT), jnp.bfloat16),
            pltpu.VMEM((M_ANCH, 1), _F32),
        ],
    )(cands, anchors, W_dec, W_enc, wts2d)
    score = jnp.reshape(score3d, (LEVELS + 1, N_CANDS))

    gold3d, acc2d, picks = pl.pallas_call(
        _finish_kernel,
        grid=(N_STRIPS,),
        in_specs=[
            pl.BlockSpec((LEVELS + 1, STRIP), lambda i: (0, i)),
            pl.BlockSpec((STRIP, D_LAT), lambda i: (i, 0)),
        ],
        out_specs=[
            pl.BlockSpec((1, 1, STRIP), lambda i: (i, 0, 0)),
            pl.BlockSpec((1, 1), lambda i: (0, 0)),
            pl.BlockSpec((STRIP, D_LAT), lambda i: (i, 0)),
        ],
        out_shape=[
            jax.ShapeDtypeStruct((N_STRIPS, 1, STRIP), jnp.int32),
            jax.ShapeDtypeStruct((1, 1), _F32),
            jax.ShapeDtypeStruct((N_CANDS, D_LAT), _F32),
        ],
    )(score, cands)

    gold = jnp.reshape(gold3d, (N_CANDS,)).astype(bool)
    acceptance = jnp.reshape(acc2d, ())
    return score, gold, acceptance, picks


# speedup vs baseline: 62.6038x; 1.5873x over previous
import jax
import jax.numpy as jnp
from jax.experimental import pallas as pl
from jax.experimental.pallas import tpu as pltpu

LEVELS = 3
K_NEIGH = 4
N_CANDS = 4096
M_ANCH = 4096
D_LAT = 128
D_DATA = 512
STRIP = 256
N_STRIPS = N_CANDS // STRIP
N_CHUNK = 32

_F32 = jnp.float32
_BF16 = jnp.bfloat16
_DN = (((1,), (0,)), ((), ()))
_DN_RHS_T = (((1,), (1,)), ((), ()))
_DN_LHS_T = (((0,), (0,)), ((), ()))


def _bfdot(a, b, dn):
    return jax.lax.dot_general(a, b, dn, preferred_element_type=_F32)


def _levels_kernel(cands_ref, anchors_ref, wd_ref, we_ref, r_ref, a_ref):
    wd_b = wd_ref[...].astype(_BF16)
    we_b = we_ref[...].astype(_BF16)
    for src_ref, dst_ref in ((cands_ref, r_ref), (anchors_ref, a_ref)):
        z = src_ref[...]
        dst_ref[0] = z
        for l in range(LEVELS):
            t = _bfdot(z.astype(_BF16), wd_b, _DN)
            z = _bfdot(t.astype(_BF16), we_b, _DN)
            dst_ref[l + 1] = z


def _score_kernel(r_lvl_ref, a_lvl_ref, aat_ref, bb_ref, wts_ref, cands_ref,
                  score_ref, gold_ref, acc_ref, picks_ref, a_bf, score_sc):
    i = pl.program_id(0)
    l = pl.program_id(1)

    @pl.when(i == 0)
    def _cast_anchors():
        a_bf[l] = a_lvl_ref[l].astype(_BF16)

    r = r_lvl_ref[0]
    nr2_bf = (r * (-2.0)).astype(_BF16)
    nab2 = _bfdot(nr2_bf, a_bf[l], _DN_RHS_T)

    onehot = (jax.lax.broadcasted_iota(jnp.int32, (1, K_NEIGH), 1) == l)
    rn_col = jnp.sum(aat_ref[...] * onehot.astype(_F32), axis=1,
                     keepdims=True)
    an_row = bb_ref[pl.ds(l, 1), :]

    big = jnp.float32(3.0e38)
    cw = M_ANCH // N_CHUNK
    m1 = jnp.full((STRIP, cw), big, _F32)
    m2, m3, m4 = m1, m1, m1
    for c in range(N_CHUNK):
        x = (rn_col + an_row[:, c * cw:(c + 1) * cw]) \
            + nab2[:, c * cw:(c + 1) * cw]
        t = jnp.minimum(m1, x); x = jnp.maximum(m1, x); m1 = t
        t = jnp.minimum(m2, x); x = jnp.maximum(m2, x); m2 = t
        t = jnp.minimum(m3, x); x = jnp.maximum(m3, x); m3 = t
        m4 = jnp.minimum(m4, x)
    st = jnp.concatenate([m1, m2, m3, m4], axis=1)

    iota = jax.lax.broadcasted_iota(jnp.int32, (STRIP, 4 * cw), 1)
    big_i = jnp.int32(2 ** 30)
    sm = []
    for k in range(K_NEIGH):
        m = jnp.min(st, axis=1, keepdims=True)
        sm.append(jnp.sqrt(jnp.maximum(m, 1e-12)).astype(_BF16))
        if k < K_NEIGH - 1:
            first = jnp.min(jnp.where(st <= m, iota, big_i),
                            axis=1, keepdims=True)
            st = jnp.where(iota == first, big, st)
    sm4 = jnp.concatenate(sm, axis=1)
    acc_col = _bfdot(sm4, wts_ref[...].astype(_BF16), _DN)

    ii = jax.lax.broadcasted_iota(jnp.int32, (STRIP, STRIP), 0)
    jj = jax.lax.broadcasted_iota(jnp.int32, (STRIP, STRIP), 1)
    eye = (ii == jj).astype(_F32)

    old = score_sc[...]
    score_sc[...] = jnp.where(onehot, acc_col, old)
    score_ref[0] = jax.lax.dot_general(
        acc_col, eye, _DN_LHS_T, preferred_element_type=_F32,
        precision=jax.lax.Precision.HIGHEST)

    @pl.when(l == LEVELS)
    def _finish():
        s4 = score_sc[...]
        g_col = jnp.logical_and(
            jnp.logical_and(s4[:, 1:2] < s4[:, 0:1], s4[:, 2:3] < s4[:, 1:2]),
            s4[:, 3:4] < s4[:, 2:3])
        gf_col = g_col.astype(_F32)
        picks_ref[...] = cands_ref[...] * gf_col
        g_row = jax.lax.dot_general(
            gf_col, eye, _DN_LHS_T, preferred_element_type=_F32,
            precision=jax.lax.Precision.HIGHEST)
        gold_ref[0] = (g_row > 0.5).astype(jnp.int32)

        @pl.when(i == 0)
        def _zero():
            acc_ref[...] = jnp.zeros((1, 1), _F32)

        acc_ref[...] += jnp.sum(gf_col, keepdims=True) / N_CANDS


def kernel(cands, anchors, W_dec, W_enc, neighbor_wts):
    r_lvl, a_lvl = pl.pallas_call(
        _levels_kernel,
        out_shape=[
            jax.ShapeDtypeStruct((LEVELS + 1, N_CANDS, D_LAT), _F32),
            jax.ShapeDtypeStruct((LEVELS + 1, M_ANCH, D_LAT), _F32),
        ],
    )(cands, anchors, W_dec, W_enc)

    aat = jnp.transpose(jnp.sum(r_lvl * r_lvl, axis=-1))
    bb = jnp.sum(a_lvl * a_lvl, axis=-1)
    wtsc = jnp.reshape(neighbor_wts, (K_NEIGH, 1))

    score3d, gold3d, acc2d, picks = pl.pallas_call(
        _score_kernel,
        grid=(N_STRIPS, LEVELS + 1),
        in_specs=[
            pl.BlockSpec((1, STRIP, D_LAT), lambda i, l: (l, i, 0)),
            pl.BlockSpec((LEVELS + 1, M_ANCH, D_LAT), lambda i, l: (0, 0, 0)),
            pl.BlockSpec((STRIP, K_NEIGH), lambda i, l: (i, 0)),
            pl.BlockSpec((LEVELS + 1, M_ANCH), lambda i, l: (0, 0)),
            pl.BlockSpec((K_NEIGH, 1), lambda i, l: (0, 0)),
            pl.BlockSpec((STRIP, D_LAT), lambda i, l: (i, 0)),
        ],
        out_specs=[
            pl.BlockSpec((1, 1, STRIP), lambda i, l: (l * N_STRIPS + i, 0, 0)),
            pl.BlockSpec((1, 1, STRIP), lambda i, l: (i, 0, 0)),
            pl.BlockSpec((1, 1), lambda i, l: (0, 0)),
            pl.BlockSpec((STRIP, D_LAT), lambda i, l: (i, 0)),
        ],
        out_shape=[
            jax.ShapeDtypeStruct(((LEVELS + 1) * N_STRIPS, 1, STRIP), _F32),
            jax.ShapeDtypeStruct((N_STRIPS, 1, STRIP), jnp.int32),
            jax.ShapeDtypeStruct((1, 1), _F32),
            jax.ShapeDtypeStruct((N_CANDS, D_LAT), _F32),
        ],
        scratch_shapes=[
            pltpu.VMEM((LEVELS + 1, M_ANCH, D_LAT), _BF16),
            pltpu.VMEM((STRIP, K_NEIGH), _F32),
        ],
    )(r_lvl, a_lvl, aat, bb, wtsc, cands)

    score = jnp.reshape(score3d, (LEVELS + 1, N_CANDS))
    gold = jnp.reshape(gold3d, (N_CANDS,)).astype(bool)
    acceptance = jnp.reshape(acc2d, ())
    return score, gold, acceptance, picks
